# baseline (speedup 1.0000x reference)
import jax
import jax.numpy as jnp
from jax import lax
from jax.experimental import pallas as pl
from jax.experimental.pallas import tpu as pltpu

_ANCHOR_SCALES = [2.0, 4.0, 8.0, 16.0, 32.0]
_ASPECT_RATIOS = [0.5, 1.0, 2.0]
_BASE_SIZE = 16.0
_PRE_NMS_TOP_N = 1000
_POST_NMS_TOP_N = 1000
_NMS_THRESH = 0.7
_MIN_BOX_SIZE = 2.0

_NCAND_PAD = 4096
_ROWS = _NCAND_PAD // 128
_PROWS = 1024 // 128


def _base_anchors_k(scale):
    import numpy as np
    out = []
    for r in _ASPECT_RATIOS:
        size = _BASE_SIZE * scale
        w = size * np.sqrt(1.0 / r)
        h = size * np.sqrt(r)
        out.append([-w / 2.0, -h / 2.0, w / 2.0, h / 2.0])
    return jnp.asarray(np.array(out, dtype=np.float32))


def _level_anchors_k(H, W, stride, scale):
    base = _base_anchors_k(scale)
    sx = jnp.arange(W, dtype=jnp.float32) * stride
    sy = jnp.arange(H, dtype=jnp.float32) * stride
    yy, xx = jnp.meshgrid(sy, sx, indexing='ij')
    shifts = jnp.stack([xx, yy, xx, yy], axis=-1).reshape(-1, 1, 4)
    return (shifts + base[None, :, :]).reshape(-1, 4)


def _conv_k(x, w, b, pad):
    y = lax.conv_general_dilated(x, w, (1, 1), pad, dimension_numbers=('NCHW', 'OIHW', 'NCHW'))
    return y + b[None, :, None, None]


def _decode_k(anchors, deltas):
    aw = anchors[:, 2] - anchors[:, 0]
    ah = anchors[:, 3] - anchors[:, 1]
    ax = anchors[:, 0] + 0.5 * aw
    ay = anchors[:, 1] + 0.5 * ah
    dx, dy, dw, dh = deltas[:, 0], deltas[:, 1], deltas[:, 2], deltas[:, 3]
    px = dx * aw + ax
    py = dy * ah + ay
    pw = jnp.exp(jnp.clip(dw, -10.0, 4.0)) * aw
    ph = jnp.exp(jnp.clip(dh, -10.0, 4.0)) * ah
    return jnp.stack([px - 0.5 * pw, py - 0.5 * ph, px + 0.5 * pw, py + 0.5 * ph], axis=1)


def _nms_pick_kernel(s_ref, x1_ref, y1_ref, x2_ref, y2_ref,
                     ox1_ref, oy1_ref, ox2_ref, oy2_ref):
    nb = s_ref.shape[0]
    x1s = [x1_ref[b] for b in range(nb)]
    y1s = [y1_ref[b] for b in range(nb)]
    x2s = [x2_ref[b] for b in range(nb)]
    y2s = [y2_ref[b] for b in range(nb)]
    areas = [jnp.maximum(x2s[b] - x1s[b], 0.0) * jnp.maximum(y2s[b] - y1s[b], 0.0)
             for b in range(nb)]
    flat = (lax.broadcasted_iota(jnp.int32, (_ROWS, 128), 0) * 128
            + lax.broadcasted_iota(jnp.int32, (_ROWS, 128), 1))
    pflat = (lax.broadcasted_iota(jnp.int32, (_PROWS, 128), 0) * 128
             + lax.broadcasted_iota(jnp.int32, (_PROWS, 128), 1))
    neg = jnp.float32(-jnp.inf)
    z = jnp.zeros((_PROWS, 128), dtype=jnp.float32)

    acts0 = tuple(s_ref[b] for b in range(nb))
    sups0 = tuple(jnp.full((_ROWS, 128), neg, dtype=jnp.float32) for _ in range(nb))
    outs0 = tuple((z, z, z, z) for _ in range(nb))

    def body(t, carry):
        acts, sups, outs = carry
        tsel = pflat == t
        new_acts = []
        new_sups = []
        new_outs = []
        for b in range(nb):
            act, sup = acts[b], sups[b]
            x1, y1, x2, y2, area = x1s[b], y1s[b], x2s[b], y2s[b], areas[b]
            m_act = jnp.max(act)
            use_act = m_act != neg
            src = jnp.where(use_act, act, sup)
            m = jnp.max(src)
            ismax = src == m
            i = jnp.min(jnp.where(ismax, flat, jnp.int32(2 ** 30)))
            sel = flat == i
            bx1 = jnp.sum(jnp.where(sel, x1, 0.0))
            by1 = jnp.sum(jnp.where(sel, y1, 0.0))
            bx2 = jnp.sum(jnp.where(sel, x2, 0.0))
            by2 = jnp.sum(jnp.where(sel, y2, 0.0))
            barea = jnp.maximum(bx2 - bx1, 0.0) * jnp.maximum(by2 - by1, 0.0)
            ix1 = jnp.maximum(x1, bx1)
            iy1 = jnp.maximum(y1, by1)
            ix2 = jnp.minimum(x2, bx2)
            iy2 = jnp.minimum(y2, by2)
            inter = jnp.maximum(ix2 - ix1, 0.0) * jnp.maximum(iy2 - iy1, 0.0)
            iou = inter / (barea + area - inter + 1e-9)
            supm = use_act & (iou > _NMS_THRESH) & (~sel) & (act != neg)
            sup = jnp.where(supm, act, sup)
            sup = jnp.where(sel & (~use_act), neg, sup)
            act = jnp.where(supm | (sel & use_act), neg, act)
            ox1, oy1, ox2, oy2 = outs[b]
            new_outs.append((jnp.where(tsel, bx1, ox1),
                             jnp.where(tsel, by1, oy1),
                             jnp.where(tsel, bx2, ox2),
                             jnp.where(tsel, by2, oy2)))
            new_acts.append(act)
            new_sups.append(sup)
        return tuple(new_acts), tuple(new_sups), tuple(new_outs)

    _, _, outs = lax.fori_loop(0, _POST_NMS_TOP_N, body, (acts0, sups0, outs0))
    for b in range(nb):
        ox1_ref[b] = outs[b][0]
        oy1_ref[b] = outs[b][1]
        ox2_ref[b] = outs[b][2]
        oy2_ref[b] = outs[b][3]


def _nms_select(boxes, scores):
    B = boxes.shape[0]
    s = scores.reshape(B, _ROWS, 128)
    fx1 = boxes[:, :, 0]
    fy1 = boxes[:, :, 1]
    fx2 = boxes[:, :, 2]
    fy2 = boxes[:, :, 3]
    x1 = fx1.reshape(B, _ROWS, 128)
    y1 = fy1.reshape(B, _ROWS, 128)
    x2 = fx2.reshape(B, _ROWS, 128)
    y2 = fy2.reshape(B, _ROWS, 128)
    in_spec = pl.BlockSpec((B, _ROWS, 128), lambda: (0, 0, 0))
    out_spec = pl.BlockSpec((B, _PROWS, 128), lambda: (0, 0, 0))
    out_sds = jax.ShapeDtypeStruct((B, _PROWS, 128), jnp.float32)
    ox1, oy1, ox2, oy2 = pl.pallas_call(
        _nms_pick_kernel,
        in_specs=[in_spec] * 5,
        out_specs=[out_spec] * 4,
        out_shape=[out_sds] * 4,
    )(s, x1, y1, x2, y2)
    n = _POST_NMS_TOP_N
    rois = jnp.stack([
        ox1.reshape(B, -1)[:, :n],
        oy1.reshape(B, -1)[:, :n],
        ox2.reshape(B, -1)[:, :n],
        oy2.reshape(B, -1)[:, :n],
    ], axis=-1)
    return rois


def kernel(feat0, feat1, feat2, feat3, feat4, im_info, W_conv, b_conv, W_cls, b_cls, W_box, b_box):
    features = [feat0, feat1, feat2, feat3, feat4]
    cls_list = []
    box_list = []
    for x in features:
        t = jax.nn.relu(_conv_k(x, W_conv, b_conv, 'SAME'))
        cls_list.append(_conv_k(t, W_cls, b_cls, 'VALID'))
        box_list.append(_conv_k(t, W_box, b_box, 'VALID'))
    scales = _ANCHOR_SCALES[::-1]
    stride = 2 ** (len(features) + 1)
    anchors_list = []
    for i, x in enumerate(features):
        anchors_list.append(_level_anchors_k(x.shape[2], x.shape[3], float(stride), scales[i]))
        stride = stride // 2
    B = features[0].shape[0]
    boxes_all = []
    scores_all = []
    for bidx in range(B):
        props = []
        scrs = []
        for lvl in range(len(features)):
            cls = cls_list[lvl][bidx]
            box = box_list[lvl][bidx]
            A, H, Wd = cls.shape[0], cls.shape[1], cls.shape[2]
            sc = jnp.transpose(cls, (1, 2, 0)).reshape(-1)
            dl = jnp.transpose(box.reshape(A, 4, H, Wd), (2, 3, 0, 1)).reshape(-1, 4)
            pb = _decode_k(anchors_list[lvl], dl)
            h_im = im_info[bidx, 0]
            w_im = im_info[bidx, 1]
            pb = jnp.stack([jnp.clip(pb[:, 0], 0.0, w_im - 1.0), jnp.clip(pb[:, 1], 0.0, h_im - 1.0), jnp.clip(pb[:, 2], 0.0, w_im - 1.0), jnp.clip(pb[:, 3], 0.0, h_im - 1.0)], axis=1)
            valid = ((pb[:, 2] - pb[:, 0]) >= _MIN_BOX_SIZE) & ((pb[:, 3] - pb[:, 1]) >= _MIN_BOX_SIZE)
            sc = jnp.where(valid, sc, -1e9)
            k = min(_PRE_NMS_TOP_N, sc.shape[0])
            top_sc, top_idx = lax.top_k(sc, k)
            props.append(pb[top_idx])
            scrs.append(top_sc)
        props = jnp.concatenate(props, axis=0)
        scrs = jnp.concatenate(scrs, axis=0)
        ncand = props.shape[0]
        pad = _NCAND_PAD - ncand
        boxes_all.append(jnp.pad(props, ((0, pad), (0, 0))))
        scores_all.append(jnp.pad(scrs, (0, pad), constant_values=-jnp.inf))
    boxes = jnp.stack(boxes_all, axis=0)
    scores = jnp.stack(scores_all, axis=0)
    rois = _nms_select(boxes, scores).reshape(B * _POST_NMS_TOP_N, 4)
    inds = jnp.repeat(jnp.arange(B, dtype=jnp.int32), _POST_NMS_TOP_N)
    return rois, inds


# speedup vs baseline: 1.1382x; 1.1382x over previous
import jax
import jax.numpy as jnp
from jax import lax
from jax.experimental import pallas as pl
from jax.experimental.pallas import tpu as pltpu

_ANCHOR_SCALES = [2.0, 4.0, 8.0, 16.0, 32.0]
_ASPECT_RATIOS = [0.5, 1.0, 2.0]
_BASE_SIZE = 16.0
_PRE_NMS_TOP_N = 1000
_POST_NMS_TOP_N = 1000
_NMS_THRESH = 0.7
_MIN_BOX_SIZE = 2.0

_NCAND_PAD = 4096
_ROWS = _NCAND_PAD // 128
_PROWS = 1024 // 128


def _base_anchors_k(scale):
    import numpy as np
    out = []
    for r in _ASPECT_RATIOS:
        size = _BASE_SIZE * scale
        w = size * np.sqrt(1.0 / r)
        h = size * np.sqrt(r)
        out.append([-w / 2.0, -h / 2.0, w / 2.0, h / 2.0])
    return jnp.asarray(np.array(out, dtype=np.float32))


def _level_anchors_k(H, W, stride, scale):
    base = _base_anchors_k(scale)
    sx = jnp.arange(W, dtype=jnp.float32) * stride
    sy = jnp.arange(H, dtype=jnp.float32) * stride
    yy, xx = jnp.meshgrid(sy, sx, indexing='ij')
    shifts = jnp.stack([xx, yy, xx, yy], axis=-1).reshape(-1, 1, 4)
    return (shifts + base[None, :, :]).reshape(-1, 4)


def _conv_k(x, w, b, pad):
    y = lax.conv_general_dilated(x, w, (1, 1), pad, dimension_numbers=('NCHW', 'OIHW', 'NCHW'))
    return y + b[None, :, None, None]


def _decode_k(anchors, deltas):
    aw = anchors[:, 2] - anchors[:, 0]
    ah = anchors[:, 3] - anchors[:, 1]
    ax = anchors[:, 0] + 0.5 * aw
    ay = anchors[:, 1] + 0.5 * ah
    dx, dy, dw, dh = deltas[:, 0], deltas[:, 1], deltas[:, 2], deltas[:, 3]
    px = dx * aw + ax
    py = dy * ah + ay
    pw = jnp.exp(jnp.clip(dw, -10.0, 4.0)) * aw
    ph = jnp.exp(jnp.clip(dh, -10.0, 4.0)) * ah
    return jnp.stack([px - 0.5 * pw, py - 0.5 * ph, px + 0.5 * pw, py + 0.5 * ph], axis=1)


def _nms_pick_kernel(s_ref, x1_ref, y1_ref, x2_ref, y2_ref,
                     sx1_ref, sy1_ref, sx2_ref, sy2_ref,
                     ox1_ref, oy1_ref, ox2_ref, oy2_ref):
    nb = s_ref.shape[0]
    x1s = [x1_ref[b] for b in range(nb)]
    y1s = [y1_ref[b] for b in range(nb)]
    x2s = [x2_ref[b] for b in range(nb)]
    y2s = [y2_ref[b] for b in range(nb)]
    areas = [jnp.maximum(x2s[b] - x1s[b], 0.0) * jnp.maximum(y2s[b] - y1s[b], 0.0)
             for b in range(nb)]
    flat = (lax.broadcasted_iota(jnp.int32, (_ROWS, 128), 0) * 128
            + lax.broadcasted_iota(jnp.int32, (_ROWS, 128), 1))
    neg = jnp.float32(-jnp.inf)

    acts0 = tuple(s_ref[b] for b in range(nb))
    sups0 = tuple(jnp.full((_ROWS, 128), neg, dtype=jnp.float32) for _ in range(nb))

    def body(t, carry):
        acts, sups = carry
        new_acts = []
        new_sups = []
        for b in range(nb):
            act, sup = acts[b], sups[b]
            x1, y1, x2, y2, area = x1s[b], y1s[b], x2s[b], y2s[b], areas[b]
            m_act = jnp.max(act)
            use_act = m_act != neg
            src = jnp.where(use_act, act, sup)
            m = jnp.max(src)
            ismax = src == m
            i = jnp.min(jnp.where(ismax, flat, jnp.int32(2 ** 30)))
            sel = flat == i
            bx1 = sx1_ref[b, i]
            by1 = sy1_ref[b, i]
            bx2 = sx2_ref[b, i]
            by2 = sy2_ref[b, i]
            barea = jnp.maximum(bx2 - bx1, 0.0) * jnp.maximum(by2 - by1, 0.0)
            ix1 = jnp.maximum(x1, bx1)
            iy1 = jnp.maximum(y1, by1)
            ix2 = jnp.minimum(x2, bx2)
            iy2 = jnp.minimum(y2, by2)
            inter = jnp.maximum(ix2 - ix1, 0.0) * jnp.maximum(iy2 - iy1, 0.0)
            iou = inter / (barea + area - inter + 1e-9)
            supm = use_act & (iou > _NMS_THRESH) & (~sel) & (act != neg)
            sup = jnp.where(supm, act, sup)
            sup = jnp.where(sel & (~use_act), neg, sup)
            act = jnp.where(supm | (sel & use_act), neg, act)
            ox1_ref[b, t] = bx1
            oy1_ref[b, t] = by1
            ox2_ref[b, t] = bx2
            oy2_ref[b, t] = by2
            new_acts.append(act)
            new_sups.append(sup)
        return tuple(new_acts), tuple(new_sups)

    lax.fori_loop(0, _POST_NMS_TOP_N, body, (acts0, sups0))


def _nms_select(boxes, scores):
    B = boxes.shape[0]
    s = scores.reshape(B, _ROWS, 128)
    fx1 = boxes[:, :, 0]
    fy1 = boxes[:, :, 1]
    fx2 = boxes[:, :, 2]
    fy2 = boxes[:, :, 3]
    x1 = fx1.reshape(B, _ROWS, 128)
    y1 = fy1.reshape(B, _ROWS, 128)
    x2 = fx2.reshape(B, _ROWS, 128)
    y2 = fy2.reshape(B, _ROWS, 128)
    in_spec = pl.BlockSpec((B, _ROWS, 128), lambda: (0, 0, 0))
    smem_spec = pl.BlockSpec((B, _NCAND_PAD), lambda: (0, 0),
                             memory_space=pltpu.SMEM)
    out_spec = pl.BlockSpec((B, 1024), lambda: (0, 0),
                            memory_space=pltpu.SMEM)
    out_sds = jax.ShapeDtypeStruct((B, 1024), jnp.float32)
    ox1, oy1, ox2, oy2 = pl.pallas_call(
        _nms_pick_kernel,
        in_specs=[in_spec] * 5 + [smem_spec] * 4,
        out_specs=[out_spec] * 4,
        out_shape=[out_sds] * 4,
    )(s, x1, y1, x2, y2, fx1, fy1, fx2, fy2)
    n = _POST_NMS_TOP_N
    rois = jnp.stack([
        ox1[:, :n],
        oy1[:, :n],
        ox2[:, :n],
        oy2[:, :n],
    ], axis=-1)
    return rois


def kernel(feat0, feat1, feat2, feat3, feat4, im_info, W_conv, b_conv, W_cls, b_cls, W_box, b_box):
    features = [feat0, feat1, feat2, feat3, feat4]
    cls_list = []
    box_list = []
    for x in features:
        t = jax.nn.relu(_conv_k(x, W_conv, b_conv, 'SAME'))
        cls_list.append(_conv_k(t, W_cls, b_cls, 'VALID'))
        box_list.append(_conv_k(t, W_box, b_box, 'VALID'))
    scales = _ANCHOR_SCALES[::-1]
    stride = 2 ** (len(features) + 1)
    anchors_list = []
    for i, x in enumerate(features):
        anchors_list.append(_level_anchors_k(x.shape[2], x.shape[3], float(stride), scales[i]))
        stride = stride // 2
    B = features[0].shape[0]
    boxes_all = []
    scores_all = []
    for bidx in range(B):
        props = []
        scrs = []
        for lvl in range(len(features)):
            cls = cls_list[lvl][bidx]
            box = box_list[lvl][bidx]
            A, H, Wd = cls.shape[0], cls.shape[1], cls.shape[2]
            sc = jnp.transpose(cls, (1, 2, 0)).reshape(-1)
            dl = jnp.transpose(box.reshape(A, 4, H, Wd), (2, 3, 0, 1)).reshape(-1, 4)
            pb = _decode_k(anchors_list[lvl], dl)
            h_im = im_info[bidx, 0]
            w_im = im_info[bidx, 1]
            pb = jnp.stack([jnp.clip(pb[:, 0], 0.0, w_im - 1.0), jnp.clip(pb[:, 1], 0.0, h_im - 1.0), jnp.clip(pb[:, 2], 0.0, w_im - 1.0), jnp.clip(pb[:, 3], 0.0, h_im - 1.0)], axis=1)
            valid = ((pb[:, 2] - pb[:, 0]) >= _MIN_BOX_SIZE) & ((pb[:, 3] - pb[:, 1]) >= _MIN_BOX_SIZE)
            sc = jnp.where(valid, sc, -1e9)
            k = min(_PRE_NMS_TOP_N, sc.shape[0])
            top_sc, top_idx = lax.top_k(sc, k)
            props.append(pb[top_idx])
            scrs.append(top_sc)
        props = jnp.concatenate(props, axis=0)
        scrs = jnp.concatenate(scrs, axis=0)
        ncand = props.shape[0]
        pad = _NCAND_PAD - ncand
        boxes_all.append(jnp.pad(props, ((0, pad), (0, 0))))
        scores_all.append(jnp.pad(scrs, (0, pad), constant_values=-jnp.inf))
    boxes = jnp.stack(boxes_all, axis=0)
    scores = jnp.stack(scores_all, axis=0)
    rois = _nms_select(boxes, scores).reshape(B * _POST_NMS_TOP_N, 4)
    inds = jnp.repeat(jnp.arange(B, dtype=jnp.int32), _POST_NMS_TOP_N)
    return rois, inds
